# baseline (speedup 1.0000x reference)
import dataclasses
import functools

import jax
import jax.numpy as jnp
from jax import lax
from jax.experimental import pallas as pl
from jax.experimental.pallas import tpu as pltpu
from jax.experimental.pallas import tpu_sc as plsc

N = 10000
E = 320000
H = 128
OUT_DIM = 64
BETA = 0.5
CVAL = 1.0

NC = 2
NS = 16
LANES = 16
NBLK = H // LANES

CHUNK = 128
TILE_CHUNKS = 84
E_PAD = NC * NS * TILE_CHUNKS * CHUNK
NROWS2 = E_PAD // CHUNK
NBUF = 3
NSLOT = 4
UNROLL = 12
TILE_ROWS = 624
ROW_TAIL = N - NS * TILE_ROWS
ROW_CHUNKS = ((0, 128), (128, 128), (256, 128), (384, 128), (512, 112))

_f32 = jnp.float32
_HIGH = lax.Precision.HIGHEST


def _sc_body(z_hbm, src_hbm, dst_hbm, w_hbm, out_hbm,
             sr0, sr1, sr2, sr3, dr0, dr1, dr2, dr3, wr0, wr1, wr2, wr3,
             r0, r1, r2, acc,
             g0, g1, g2, s0, s1, s2, e0, e1, e2, e3):
    cid = lax.axis_index("c")
    sid = lax.axis_index("s")
    srings = (sr0, sr1, sr2, sr3)
    drings = (dr0, dr1, dr2, dr3)
    wrings = (wr0, wr1, wr2, wr3)
    bufs = (r0, r1, r2)
    gsems = (g0, g1, g2)
    ssems = (s0, s1, s2)
    esems = (e0, e1, e2, e3)

    @pl.loop(0, CHUNK)
    def _zero(i):
        for b in range(NBLK):
            r0[i, pl.ds(b * LANES, LANES)] = jnp.zeros((LANES,), _f32)

    row0 = pl.multiple_of(sid * TILE_ROWS, 8)
    for off, sz in ROW_CHUNKS:
        pltpu.sync_copy(r0.at[pl.ds(0, sz)],
                        acc.at[pl.ds(row0 + off, sz)])

    @pl.when(sid == 0)
    def _zero_tail():
        pltpu.sync_copy(r0.at[pl.ds(0, ROW_TAIL)],
                        acc.at[pl.ds(NS * TILE_ROWS, ROW_TAIL)])

    plsc.subcore_barrier()

    ebase = pl.multiple_of((cid * NS + sid) * TILE_CHUNKS * CHUNK, 8)

    def estart(c, sl):
        off = ebase + c * CHUNK
        pltpu.async_copy(src_hbm.at[pl.ds(off, CHUNK)], srings[sl], esems[sl])
        pltpu.async_copy(dst_hbm.at[pl.ds(off, CHUNK)], drings[sl], esems[sl])
        pltpu.async_copy(w_hbm.at[pl.ds(off, CHUNK)], wrings[sl], esems[sl])

    def ewait(sl):
        pltpu.make_async_copy(src_hbm.at[pl.ds(0, CHUNK)], srings[sl],
                              esems[sl]).wait()
        pltpu.make_async_copy(dst_hbm.at[pl.ds(0, CHUNK)], drings[sl],
                              esems[sl]).wait()
        pltpu.make_async_copy(w_hbm.at[pl.ds(0, CHUNK)], wrings[sl],
                              esems[sl]).wait()

    def gstart(sl, b):
        pltpu.async_copy(z_hbm.at[srings[sl]], bufs[b], gsems[b])

    def gwait(b):
        pltpu.make_async_copy(z_hbm.at[srings[0]], bufs[b], gsems[b]).wait()

    def sstart(sl, b):
        pltpu.async_copy(bufs[b], acc.at[drings[sl]], ssems[b], add=True)

    def swait(b):
        pltpu.make_async_copy(bufs[b], acc.at[drings[0]], ssems[b]).wait()

    def scale(sl, b):
        buf = bufs[b]
        wv = wrings[sl]

        @plsc.parallel_loop(0, CHUNK, unroll=8)
        def _(e):
            wb = plsc.load_gather(wv, [jnp.full((LANES,), e, jnp.int32)])
            for blk in range(NBLK):
                s = pl.ds(blk * LANES, LANES)
                buf[e, s] = buf[e, s] * wb

    for c in range(NBUF):
        off = ebase + c * CHUNK
        pltpu.sync_copy(src_hbm.at[pl.ds(off, CHUNK)], srings[c])
        pltpu.sync_copy(dst_hbm.at[pl.ds(off, CHUNK)], drings[c])
        pltpu.sync_copy(w_hbm.at[pl.ds(off, CHUNK)], wrings[c])
    gstart(0, 0)
    gstart(1, 1)

    @pl.loop(0, TILE_CHUNKS, step=UNROLL)
    def _edges(j):
        for u in range(UNROLL):
            jj = j + u
            b = u % NBUF
            sl = u % NSLOT
            b2 = (u + 2) % NBUF
            sl2 = (u + 2) % NSLOT
            sl3 = (u + 3) % NSLOT

            gwait(b)
            scale(sl, b)
            sstart(sl, b)

            @pl.when(jnp.logical_and(jj >= 1, jj + 2 < TILE_CHUNKS))
            def _():
                swait(b2)

            @pl.when(jnp.logical_and(jj + 2 >= NBUF, jj + 2 < TILE_CHUNKS))
            def _():
                ewait(sl2)

            @pl.when(jj + 2 < TILE_CHUNKS)
            def _():
                gstart(sl2, b2)

            @pl.when(jj + 3 < TILE_CHUNKS)
            def _():
                estart(jj + 3, sl3)

    for b in range(NBUF):
        swait(b)

    plsc.subcore_barrier()

    for off, sz in ROW_CHUNKS:
        sl = pl.ds(row0 + off, sz)
        pltpu.sync_copy(acc.at[sl], out_hbm.at[cid].at[sl])

    @pl.when(sid == 0)
    def _drain_tail():
        sl = pl.ds(NS * TILE_ROWS, ROW_TAIL)
        pltpu.sync_copy(acc.at[sl], out_hbm.at[cid].at[sl])


def _sc_params():
    cp = pltpu.CompilerParams()
    if "needs_layout_passes" in pltpu.CompilerParams.__dataclass_fields__:
        cp = dataclasses.replace(cp, needs_layout_passes=False)
    return cp


def _sc_agg(z, src, dst, w):
    mesh = plsc.VectorSubcoreMesh(core_axis_name="c", subcore_axis_name="s",
                                  num_cores=NC, num_subcores=NS)
    kfn = pl.kernel(
        _sc_body,
        out_type=jax.ShapeDtypeStruct((NC, N, H), _f32),
        mesh=mesh,
        scratch_types=(
            [pltpu.VMEM((CHUNK,), jnp.int32) for _ in range(2 * NSLOT)]
            + [pltpu.VMEM((CHUNK,), _f32) for _ in range(NSLOT)]
            + [pltpu.VMEM((CHUNK, H), _f32) for _ in range(NBUF)]
            + [pltpu.VMEM_SHARED((N, H), _f32)]
            + [pltpu.SemaphoreType.DMA for _ in range(2 * NBUF + NSLOT)]
        ),
        compiler_params=_sc_params(),
    )
    return kfn(z, src, dst, w)


BR = 1000
_GRID = N // BR


def _row_spec(width):
    return pl.BlockSpec((BR, width), lambda i: (i, 0))


def _full_spec(shape):
    return pl.BlockSpec(shape, lambda i: tuple(0 for _ in shape))


def _tc_first(x, W0):
    def body(x_ref, w_ref, o_ref):
        o_ref[...] = jnp.dot(x_ref[...], w_ref[...], precision=_HIGH,
                             preferred_element_type=_f32)
    return pl.pallas_call(
        body,
        grid=(_GRID,),
        in_specs=[_row_spec(H), _full_spec((H, H))],
        out_specs=_row_spec(H),
        out_shape=jax.ShapeDtypeStruct((N, H), _f32))(x, W0)


def _tc_mid(p, b, Wn):
    def body(p_ref, b_ref, w_ref, h_ref, z_ref):
        zagg = p_ref[0] + p_ref[1] + b_ref[...]
        h = BETA * zagg + (CVAL - BETA) * jnp.maximum(zagg, 0.0)
        h_ref[...] = h
        z_ref[...] = jnp.dot(h, w_ref[...], precision=_HIGH,
                             preferred_element_type=_f32)
    return pl.pallas_call(
        body,
        grid=(_GRID,),
        in_specs=[pl.BlockSpec((NC, BR, H), lambda i: (0, i, 0)),
                  _full_spec((1, H)), _full_spec((H, H))],
        out_specs=(_row_spec(H), _row_spec(H)),
        out_shape=(jax.ShapeDtypeStruct((N, H), _f32),
                   jax.ShapeDtypeStruct((N, H), _f32)),
    )(p, b.reshape(1, H), Wn)


def _tc_final(p, b2, h0, h1, Wlin, blin):
    def body(p_ref, b_ref, h0_ref, h1_ref, wl_ref, bl_ref, o_ref):
        zagg = p_ref[0] + p_ref[1] + b_ref[...]
        h2 = BETA * zagg + (CVAL - BETA) * jnp.maximum(zagg, 0.0)
        o_ref[...] = (
            jnp.dot(h0_ref[...], wl_ref[0:H], precision=_HIGH,
                    preferred_element_type=_f32)
            + jnp.dot(h1_ref[...], wl_ref[H:2 * H], precision=_HIGH,
                      preferred_element_type=_f32)
            + jnp.dot(h2, wl_ref[2 * H:3 * H], precision=_HIGH,
                      preferred_element_type=_f32)
            + bl_ref[...])
    return pl.pallas_call(
        body,
        grid=(_GRID,),
        in_specs=[pl.BlockSpec((NC, BR, H), lambda i: (0, i, 0)),
                  _full_spec((1, H)), _row_spec(H), _row_spec(H),
                  _full_spec((3 * H, OUT_DIM)), _full_spec((OUT_DIM,))],
        out_specs=_row_spec(OUT_DIM),
        out_shape=jax.ShapeDtypeStruct((N, OUT_DIM), _f32),
    )(p, b2.reshape(1, H), h0, h1, Wlin, blin)


def kernel(x, edge_index, edge_weight, W0, b0, W1, b1, W2, b2, Wlin, blin):
    pad = E_PAD - E
    spread = (jnp.arange(pad, dtype=jnp.int32) * 97) % N
    src = jnp.concatenate([edge_index[0], spread])
    dst = jnp.concatenate([edge_index[1], spread])
    edge_weight = jnp.concatenate([edge_weight, jnp.zeros((pad,), _f32)])
    z = _tc_first(x, W0)
    p = _sc_agg(z, src, dst, edge_weight)
    h0, z = _tc_mid(p, b0, W1)
    p = _sc_agg(z, src, dst, edge_weight)
    h1, z = _tc_mid(p, b1, W2)
    p = _sc_agg(z, src, dst, edge_weight)
    return _tc_final(p, b2, h0, h1, Wlin, blin)


# speedup vs baseline: 1.0145x; 1.0145x over previous
import dataclasses
import functools

import jax
import jax.numpy as jnp
from jax import lax
from jax.experimental import pallas as pl
from jax.experimental.pallas import tpu as pltpu
from jax.experimental.pallas import tpu_sc as plsc

N = 10000
E = 320000
H = 128
OUT_DIM = 64
BETA = 0.5
CVAL = 1.0

NC = 2
NS = 16
LANES = 16
NBLK = H // LANES

CHUNK = 128
TILE_CHUNKS = 84
E_PAD = NC * NS * TILE_CHUNKS * CHUNK
NROWS2 = E_PAD // CHUNK
NBUF = 3
NSLOT = 4
UNROLL = 12
TILE_ROWS = 624
ROW_TAIL = N - NS * TILE_ROWS
ROW_CHUNKS = ((0, 128), (128, 128), (256, 128), (384, 128), (512, 112))

_f32 = jnp.float32
_HIGH = lax.Precision.HIGHEST


def _sc_body(z_hbm, src_hbm, dst_hbm, w_hbm, out_hbm,
             sr0, sr1, sr2, sr3, dr0, dr1, dr2, dr3, wr0, wr1, wr2, wr3,
             r0, r1, r2, acc,
             g0, g1, g2, s0, s1, s2, e0, e1, e2, e3):
    cid = lax.axis_index("c")
    sid = lax.axis_index("s")
    srings = (sr0, sr1, sr2, sr3)
    drings = (dr0, dr1, dr2, dr3)
    wrings = (wr0, wr1, wr2, wr3)
    bufs = (r0, r1, r2)
    gsems = (g0, g1, g2)
    ssems = (s0, s1, s2)
    esems = (e0, e1, e2, e3)

    @pl.loop(0, CHUNK)
    def _zero(i):
        for b in range(NBLK):
            r0[i, pl.ds(b * LANES, LANES)] = jnp.zeros((LANES,), _f32)

    row0 = pl.multiple_of(sid * TILE_ROWS, 8)
    for off, sz in ROW_CHUNKS:
        pltpu.sync_copy(r0.at[pl.ds(0, sz)],
                        acc.at[pl.ds(row0 + off, sz)])

    @pl.when(sid == 0)
    def _zero_tail():
        pltpu.sync_copy(r0.at[pl.ds(0, ROW_TAIL)],
                        acc.at[pl.ds(NS * TILE_ROWS, ROW_TAIL)])

    plsc.subcore_barrier()

    ebase = pl.multiple_of((cid * NS + sid) * TILE_CHUNKS * CHUNK, 8)

    def estart(c, sl):
        off = ebase + c * CHUNK
        pltpu.async_copy(src_hbm.at[pl.ds(off, CHUNK)], srings[sl], esems[sl])
        pltpu.async_copy(dst_hbm.at[pl.ds(off, CHUNK)], drings[sl], esems[sl])
        pltpu.async_copy(w_hbm.at[pl.ds(off, CHUNK)], wrings[sl], esems[sl])

    def ewait(sl):
        pltpu.make_async_copy(src_hbm.at[pl.ds(0, CHUNK)], srings[sl],
                              esems[sl]).wait()
        pltpu.make_async_copy(dst_hbm.at[pl.ds(0, CHUNK)], drings[sl],
                              esems[sl]).wait()
        pltpu.make_async_copy(w_hbm.at[pl.ds(0, CHUNK)], wrings[sl],
                              esems[sl]).wait()

    def gstart(sl, b):
        pltpu.async_copy(z_hbm.at[srings[sl]], bufs[b], gsems[b])

    def gwait(b):
        pltpu.make_async_copy(z_hbm.at[srings[0]], bufs[b], gsems[b]).wait()

    def sstart(sl, b):
        pltpu.async_copy(bufs[b], acc.at[drings[sl]], ssems[b], add=True)

    def swait(b):
        pltpu.make_async_copy(bufs[b], acc.at[drings[0]], ssems[b]).wait()

    def scale(sl, b):
        buf = bufs[b]
        wv = wrings[sl]

        @plsc.parallel_loop(0, CHUNK, unroll=4)
        def _(e):
            wb = plsc.load_gather(wv, [jnp.full((LANES,), e, jnp.int32)])
            for blk in range(NBLK):
                s = pl.ds(blk * LANES, LANES)
                buf[e, s] = buf[e, s] * wb

    for c in range(NBUF):
        off = ebase + c * CHUNK
        pltpu.sync_copy(src_hbm.at[pl.ds(off, CHUNK)], srings[c])
        pltpu.sync_copy(dst_hbm.at[pl.ds(off, CHUNK)], drings[c])
        pltpu.sync_copy(w_hbm.at[pl.ds(off, CHUNK)], wrings[c])
    gstart(0, 0)
    gstart(1, 1)

    @pl.loop(0, TILE_CHUNKS, step=UNROLL)
    def _edges(j):
        for u in range(UNROLL):
            jj = j + u
            b = u % NBUF
            sl = u % NSLOT
            b2 = (u + 2) % NBUF
            sl2 = (u + 2) % NSLOT
            sl3 = (u + 3) % NSLOT

            gwait(b)
            scale(sl, b)
            sstart(sl, b)

            @pl.when(jnp.logical_and(jj >= 1, jj + 2 < TILE_CHUNKS))
            def _():
                swait(b2)

            @pl.when(jnp.logical_and(jj + 2 >= NBUF, jj + 2 < TILE_CHUNKS))
            def _():
                ewait(sl2)

            @pl.when(jj + 2 < TILE_CHUNKS)
            def _():
                gstart(sl2, b2)

            @pl.when(jj + 3 < TILE_CHUNKS)
            def _():
                estart(jj + 3, sl3)

    for b in range(NBUF):
        swait(b)

    plsc.subcore_barrier()

    for off, sz in ROW_CHUNKS:
        sl = pl.ds(row0 + off, sz)
        pltpu.sync_copy(acc.at[sl], out_hbm.at[cid].at[sl])

    @pl.when(sid == 0)
    def _drain_tail():
        sl = pl.ds(NS * TILE_ROWS, ROW_TAIL)
        pltpu.sync_copy(acc.at[sl], out_hbm.at[cid].at[sl])


def _sc_params():
    cp = pltpu.CompilerParams()
    if "needs_layout_passes" in pltpu.CompilerParams.__dataclass_fields__:
        cp = dataclasses.replace(cp, needs_layout_passes=False)
    return cp


def _sc_agg(z, src, dst, w):
    mesh = plsc.VectorSubcoreMesh(core_axis_name="c", subcore_axis_name="s",
                                  num_cores=NC, num_subcores=NS)
    kfn = pl.kernel(
        _sc_body,
        out_type=jax.ShapeDtypeStruct((NC, N, H), _f32),
        mesh=mesh,
        scratch_types=(
            [pltpu.VMEM((CHUNK,), jnp.int32) for _ in range(2 * NSLOT)]
            + [pltpu.VMEM((CHUNK,), _f32) for _ in range(NSLOT)]
            + [pltpu.VMEM((CHUNK, H), _f32) for _ in range(NBUF)]
            + [pltpu.VMEM_SHARED((N, H), _f32)]
            + [pltpu.SemaphoreType.DMA for _ in range(2 * NBUF + NSLOT)]
        ),
        compiler_params=_sc_params(),
    )
    return kfn(z, src, dst, w)


BR = 1000
_GRID = N // BR


def _row_spec(width):
    return pl.BlockSpec((BR, width), lambda i: (i, 0))


def _full_spec(shape):
    return pl.BlockSpec(shape, lambda i: tuple(0 for _ in shape))


def _tc_first(x, W0):
    def body(x_ref, w_ref, o_ref):
        o_ref[...] = jnp.dot(x_ref[...], w_ref[...], precision=_HIGH,
                             preferred_element_type=_f32)
    return pl.pallas_call(
        body,
        grid=(_GRID,),
        in_specs=[_row_spec(H), _full_spec((H, H))],
        out_specs=_row_spec(H),
        out_shape=jax.ShapeDtypeStruct((N, H), _f32))(x, W0)


def _tc_mid(p, b, Wn):
    def body(p_ref, b_ref, w_ref, h_ref, z_ref):
        zagg = p_ref[0] + p_ref[1] + b_ref[...]
        h = BETA * zagg + (CVAL - BETA) * jnp.maximum(zagg, 0.0)
        h_ref[...] = h
        z_ref[...] = jnp.dot(h, w_ref[...], precision=_HIGH,
                             preferred_element_type=_f32)
    return pl.pallas_call(
        body,
        grid=(_GRID,),
        in_specs=[pl.BlockSpec((NC, BR, H), lambda i: (0, i, 0)),
                  _full_spec((1, H)), _full_spec((H, H))],
        out_specs=(_row_spec(H), _row_spec(H)),
        out_shape=(jax.ShapeDtypeStruct((N, H), _f32),
                   jax.ShapeDtypeStruct((N, H), _f32)),
    )(p, b.reshape(1, H), Wn)


def _tc_final(p, b2, h0, h1, Wlin, blin):
    def body(p_ref, b_ref, h0_ref, h1_ref, wl_ref, bl_ref, o_ref):
        zagg = p_ref[0] + p_ref[1] + b_ref[...]
        h2 = BETA * zagg + (CVAL - BETA) * jnp.maximum(zagg, 0.0)
        o_ref[...] = (
            jnp.dot(h0_ref[...], wl_ref[0:H], precision=_HIGH,
                    preferred_element_type=_f32)
            + jnp.dot(h1_ref[...], wl_ref[H:2 * H], precision=_HIGH,
                      preferred_element_type=_f32)
            + jnp.dot(h2, wl_ref[2 * H:3 * H], precision=_HIGH,
                      preferred_element_type=_f32)
            + bl_ref[...])
    return pl.pallas_call(
        body,
        grid=(_GRID,),
        in_specs=[pl.BlockSpec((NC, BR, H), lambda i: (0, i, 0)),
                  _full_spec((1, H)), _row_spec(H), _row_spec(H),
                  _full_spec((3 * H, OUT_DIM)), _full_spec((OUT_DIM,))],
        out_specs=_row_spec(OUT_DIM),
        out_shape=jax.ShapeDtypeStruct((N, OUT_DIM), _f32),
    )(p, b2.reshape(1, H), h0, h1, Wlin, blin)


def kernel(x, edge_index, edge_weight, W0, b0, W1, b1, W2, b2, Wlin, blin):
    pad = E_PAD - E
    spread = (jnp.arange(pad, dtype=jnp.int32) * 97) % N
    src = jnp.concatenate([edge_index[0], spread])
    dst = jnp.concatenate([edge_index[1], spread])
    edge_weight = jnp.concatenate([edge_weight, jnp.zeros((pad,), _f32)])
    z = _tc_first(x, W0)
    p = _sc_agg(z, src, dst, edge_weight)
    h0, z = _tc_mid(p, b0, W1)
    p = _sc_agg(z, src, dst, edge_weight)
    h1, z = _tc_mid(p, b1, W2)
    p = _sc_agg(z, src, dst, edge_weight)
    return _tc_final(p, b2, h0, h1, Wlin, blin)


# speedup vs baseline: 1.0289x; 1.0143x over previous
import dataclasses
import functools

import jax
import jax.numpy as jnp
from jax import lax
from jax.experimental import pallas as pl
from jax.experimental.pallas import tpu as pltpu
from jax.experimental.pallas import tpu_sc as plsc

N = 10000
E = 320000
H = 128
OUT_DIM = 64
BETA = 0.5
CVAL = 1.0

NC = 2
NS = 16
LANES = 16
NBLK = H // LANES

CHUNK = 128
TILE_CHUNKS = 84
E_PAD = NC * NS * TILE_CHUNKS * CHUNK
NROWS2 = E_PAD // CHUNK
NBUF = 3
NSLOT = 4
UNROLL = 12
TILE_ROWS = 624
ROW_TAIL = N - NS * TILE_ROWS
ROW_CHUNKS = ((0, 128), (128, 128), (256, 128), (384, 128), (512, 112))

_f32 = jnp.float32
_HIGH = lax.Precision.HIGHEST


def _sc_body(z_hbm, src_hbm, dst_hbm, w_hbm, out_hbm,
             sr0, sr1, sr2, sr3, dr0, dr1, dr2, dr3, wr0, wr1, wr2, wr3,
             r0, r1, r2, acc,
             g0, g1, g2, s0, s1, s2, e0, e1, e2, e3, zsem):
    cid = lax.axis_index("c")
    sid = lax.axis_index("s")
    srings = (sr0, sr1, sr2, sr3)
    drings = (dr0, dr1, dr2, dr3)
    wrings = (wr0, wr1, wr2, wr3)
    bufs = (r0, r1, r2)
    gsems = (g0, g1, g2)
    ssems = (s0, s1, s2)
    esems = (e0, e1, e2, e3)

    @pl.loop(0, CHUNK)
    def _zero(i):
        for b in range(NBLK):
            r0[i, pl.ds(b * LANES, LANES)] = jnp.zeros((LANES,), _f32)

    row0 = pl.multiple_of(sid * TILE_ROWS, 8)
    for off, sz in ROW_CHUNKS:
        pltpu.async_copy(r0.at[pl.ds(0, sz)],
                         acc.at[pl.ds(row0 + off, sz)], zsem)

    @pl.when(sid == 0)
    def _zero_tail():
        pltpu.async_copy(r0.at[pl.ds(0, ROW_TAIL)],
                         acc.at[pl.ds(NS * TILE_ROWS, ROW_TAIL)], zsem)

    ebase = pl.multiple_of((cid * NS + sid) * TILE_CHUNKS * CHUNK, 8)

    def estart(c, sl):
        off = ebase + c * CHUNK
        pltpu.async_copy(src_hbm.at[pl.ds(off, CHUNK)], srings[sl], esems[sl])
        pltpu.async_copy(dst_hbm.at[pl.ds(off, CHUNK)], drings[sl], esems[sl])
        pltpu.async_copy(w_hbm.at[pl.ds(off, CHUNK)], wrings[sl], esems[sl])

    def ewait(sl):
        pltpu.make_async_copy(src_hbm.at[pl.ds(0, CHUNK)], srings[sl],
                              esems[sl]).wait()
        pltpu.make_async_copy(dst_hbm.at[pl.ds(0, CHUNK)], drings[sl],
                              esems[sl]).wait()
        pltpu.make_async_copy(w_hbm.at[pl.ds(0, CHUNK)], wrings[sl],
                              esems[sl]).wait()

    def gstart(sl, b):
        pltpu.async_copy(z_hbm.at[srings[sl]], bufs[b], gsems[b])

    def gwait(b):
        pltpu.make_async_copy(z_hbm.at[srings[0]], bufs[b], gsems[b]).wait()

    def sstart(sl, b):
        pltpu.async_copy(bufs[b], acc.at[drings[sl]], ssems[b], add=True)

    def swait(b):
        pltpu.make_async_copy(bufs[b], acc.at[drings[0]], ssems[b]).wait()

    def scale(sl, b):
        buf = bufs[b]
        wv = wrings[sl]

        @plsc.parallel_loop(0, CHUNK, unroll=4)
        def _(e):
            wb = plsc.load_gather(wv, [jnp.full((LANES,), e, jnp.int32)])
            for blk in range(NBLK):
                s = pl.ds(blk * LANES, LANES)
                buf[e, s] = buf[e, s] * wb

    for c in range(NBUF):
        off = ebase + c * CHUNK
        pltpu.sync_copy(src_hbm.at[pl.ds(off, CHUNK)], srings[c])
        pltpu.sync_copy(dst_hbm.at[pl.ds(off, CHUNK)], drings[c])
        pltpu.sync_copy(w_hbm.at[pl.ds(off, CHUNK)], wrings[c])

    for off, sz in ROW_CHUNKS:
        pltpu.make_async_copy(r0.at[pl.ds(0, sz)],
                              acc.at[pl.ds(row0 + off, sz)], zsem).wait()

    @pl.when(sid == 0)
    def _zero_tail_wait():
        pltpu.make_async_copy(r0.at[pl.ds(0, ROW_TAIL)],
                              acc.at[pl.ds(NS * TILE_ROWS, ROW_TAIL)],
                              zsem).wait()

    gstart(0, 0)
    gstart(1, 1)

    plsc.subcore_barrier()

    @pl.loop(0, TILE_CHUNKS, step=UNROLL)
    def _edges(j):
        for u in range(UNROLL):
            jj = j + u
            b = u % NBUF
            sl = u % NSLOT
            b2 = (u + 2) % NBUF
            sl2 = (u + 2) % NSLOT
            sl3 = (u + 3) % NSLOT

            gwait(b)
            scale(sl, b)
            sstart(sl, b)

            @pl.when(jnp.logical_and(jj >= 1, jj + 2 < TILE_CHUNKS))
            def _():
                swait(b2)

            @pl.when(jnp.logical_and(jj + 2 >= NBUF, jj + 2 < TILE_CHUNKS))
            def _():
                ewait(sl2)

            @pl.when(jj + 2 < TILE_CHUNKS)
            def _():
                gstart(sl2, b2)

            @pl.when(jj + 3 < TILE_CHUNKS)
            def _():
                estart(jj + 3, sl3)

    for b in range(NBUF):
        swait(b)

    plsc.subcore_barrier()

    for off, sz in ROW_CHUNKS:
        sl = pl.ds(row0 + off, sz)
        pltpu.async_copy(acc.at[sl], out_hbm.at[cid].at[sl], zsem)

    @pl.when(sid == 0)
    def _drain_tail():
        sl = pl.ds(NS * TILE_ROWS, ROW_TAIL)
        pltpu.async_copy(acc.at[sl], out_hbm.at[cid].at[sl], zsem)

    for off, sz in ROW_CHUNKS:
        sl = pl.ds(row0 + off, sz)
        pltpu.make_async_copy(acc.at[sl], out_hbm.at[cid].at[sl], zsem).wait()

    @pl.when(sid == 0)
    def _drain_tail_wait():
        sl = pl.ds(NS * TILE_ROWS, ROW_TAIL)
        pltpu.make_async_copy(acc.at[sl], out_hbm.at[cid].at[sl],
                              zsem).wait()


def _sc_params():
    cp = pltpu.CompilerParams()
    if "needs_layout_passes" in pltpu.CompilerParams.__dataclass_fields__:
        cp = dataclasses.replace(cp, needs_layout_passes=False)
    return cp


def _sc_agg(z, src, dst, w):
    mesh = plsc.VectorSubcoreMesh(core_axis_name="c", subcore_axis_name="s",
                                  num_cores=NC, num_subcores=NS)
    kfn = pl.kernel(
        _sc_body,
        out_type=jax.ShapeDtypeStruct((NC, N, H), _f32),
        mesh=mesh,
        scratch_types=(
            [pltpu.VMEM((CHUNK,), jnp.int32) for _ in range(2 * NSLOT)]
            + [pltpu.VMEM((CHUNK,), _f32) for _ in range(NSLOT)]
            + [pltpu.VMEM((CHUNK, H), _f32) for _ in range(NBUF)]
            + [pltpu.VMEM_SHARED((N, H), _f32)]
            + [pltpu.SemaphoreType.DMA for _ in range(2 * NBUF + NSLOT + 1)]
        ),
        compiler_params=_sc_params(),
    )
    return kfn(z, src, dst, w)


BR = 1000
_GRID = N // BR


def _row_spec(width):
    return pl.BlockSpec((BR, width), lambda i: (i, 0))


def _full_spec(shape):
    return pl.BlockSpec(shape, lambda i: tuple(0 for _ in shape))


def _tc_first(x, W0):
    def body(x_ref, w_ref, o_ref):
        o_ref[...] = jnp.dot(x_ref[...], w_ref[...], precision=_HIGH,
                             preferred_element_type=_f32)
    return pl.pallas_call(
        body,
        grid=(_GRID,),
        in_specs=[_row_spec(H), _full_spec((H, H))],
        out_specs=_row_spec(H),
        out_shape=jax.ShapeDtypeStruct((N, H), _f32))(x, W0)


def _tc_mid(p, b, Wn):
    def body(p_ref, b_ref, w_ref, h_ref, z_ref):
        zagg = p_ref[0] + p_ref[1] + b_ref[...]
        h = BETA * zagg + (CVAL - BETA) * jnp.maximum(zagg, 0.0)
        h_ref[...] = h
        z_ref[...] = jnp.dot(h, w_ref[...], precision=_HIGH,
                             preferred_element_type=_f32)
    return pl.pallas_call(
        body,
        grid=(_GRID,),
        in_specs=[pl.BlockSpec((NC, BR, H), lambda i: (0, i, 0)),
                  _full_spec((1, H)), _full_spec((H, H))],
        out_specs=(_row_spec(H), _row_spec(H)),
        out_shape=(jax.ShapeDtypeStruct((N, H), _f32),
                   jax.ShapeDtypeStruct((N, H), _f32)),
    )(p, b.reshape(1, H), Wn)


def _tc_final(p, b2, h0, h1, Wlin, blin):
    def body(p_ref, b_ref, h0_ref, h1_ref, wl_ref, bl_ref, o_ref):
        zagg = p_ref[0] + p_ref[1] + b_ref[...]
        h2 = BETA * zagg + (CVAL - BETA) * jnp.maximum(zagg, 0.0)
        o_ref[...] = (
            jnp.dot(h0_ref[...], wl_ref[0:H], precision=_HIGH,
                    preferred_element_type=_f32)
            + jnp.dot(h1_ref[...], wl_ref[H:2 * H], precision=_HIGH,
                      preferred_element_type=_f32)
            + jnp.dot(h2, wl_ref[2 * H:3 * H], precision=_HIGH,
                      preferred_element_type=_f32)
            + bl_ref[...])
    return pl.pallas_call(
        body,
        grid=(_GRID,),
        in_specs=[pl.BlockSpec((NC, BR, H), lambda i: (0, i, 0)),
                  _full_spec((1, H)), _row_spec(H), _row_spec(H),
                  _full_spec((3 * H, OUT_DIM)), _full_spec((OUT_DIM,))],
        out_specs=_row_spec(OUT_DIM),
        out_shape=jax.ShapeDtypeStruct((N, OUT_DIM), _f32),
    )(p, b2.reshape(1, H), h0, h1, Wlin, blin)


def kernel(x, edge_index, edge_weight, W0, b0, W1, b1, W2, b2, Wlin, blin):
    pad = E_PAD - E
    spread = (jnp.arange(pad, dtype=jnp.int32) * 97) % N
    src = jnp.concatenate([edge_index[0], spread])
    dst = jnp.concatenate([edge_index[1], spread])
    edge_weight = jnp.concatenate([edge_weight, jnp.zeros((pad,), _f32)])
    z = _tc_first(x, W0)
    p = _sc_agg(z, src, dst, edge_weight)
    h0, z = _tc_mid(p, b0, W1)
    p = _sc_agg(z, src, dst, edge_weight)
    h1, z = _tc_mid(p, b1, W2)
    p = _sc_agg(z, src, dst, edge_weight)
    return _tc_final(p, b2, h0, h1, Wlin, blin)
